# baseline (speedup 1.0000x reference)
import functools

import jax
import jax.numpy as jnp
from jax import lax
from jax.experimental import pallas as pl
from jax.experimental.pallas import tpu as pltpu
from jax.experimental.pallas import tpu_sc as plsc

D = 256
DD = 512
MV_CHUNK = 2048
MV_STEPS = 8


def _ohf(x, n):
    i = lax.broadcasted_iota(jnp.int32, (x.shape[0], n), 1)
    return (i == x).astype(jnp.float32)


def _bitsf(x, nbits):
    i = lax.broadcasted_iota(jnp.int32, (x.shape[0], nbits), 1)
    return (jnp.bitwise_and(x, jnp.left_shift(jnp.int32(1), i)) != 0
            ).astype(jnp.float32)


def _fuse_body(st_ref, ab_ref, it_ref, mv_ref,
               whp_ref, wlv_ref, wac_ref, woh_ref,
               wsp_ref, wab_ref, wit_ref, wmv_ref,
               bhp_ref, blv_ref, bac_ref, boh_ref,
               bsp_ref, bab_ref, bit_ref, bmv_ref,
               wg_ref, wv_ref,
               t_hp, t_lv, t_gsbtn, t_tsfa, t_sp, t_ab, t_itfx, t_mvpp, t_r,
               mv_f, pp_blk):
    step = pl.program_id(0)

    def dot(a, b):
        return jnp.dot(a, b, preferred_element_type=jnp.float32,
                       precision=lax.Precision.HIGHEST)

    def mcat(i):
        return jnp.concatenate([wg_ref[i], wv_ref[i]], axis=1)

    @pl.when(step == 0)
    def _():
        m0 = mcat(0)
        m4 = mcat(4)
        b0 = (bhp_ref[...] + blv_ref[...] + bac_ref[...] + boh_ref[...])
        bias0 = dot(b0, m0)

        r10 = lax.broadcasted_iota(jnp.int32, (1024, 1), 0)
        t_hp[...] = dot(_bitsf(r10, 10), dot(whp_ref[...], m0)) + bias0
        r7 = lax.broadcasted_iota(jnp.int32, (128, 1), 0)
        t_lv[...] = dot(_bitsf(r7, 7), dot(wlv_ref[...], m0))

        m31 = dot(woh_ref[...], m0)
        r = lax.broadcasted_iota(jnp.int32, (864, 1), 0)
        e1 = jnp.concatenate([
            jnp.zeros((864, 1), jnp.float32),
            _ohf(r // 216, 3), _ohf((r // 27) % 8, 7),
            _ohf((r // 9) % 3, 2), _ohf((r // 3) % 3, 2), _ohf(r % 3, 2),
            jnp.zeros((864, 14), jnp.float32)], axis=1)
        t_gsbtn[...] = dot(e1, m31)

        r = lax.broadcasted_iota(jnp.int32, (405, 1), 0)
        e2 = jnp.concatenate([
            jnp.zeros((405, 17), jnp.float32),
            _ohf(r // 45, 8), _ohf((r // 9) % 5, 4), _ohf((r // 3) % 3, 2),
        ], axis=1)
        t_tsfa[...] = dot(e2, m31) + dot(_ohf(r % 3, 2), dot(wac_ref[...], m0))

        m1 = mcat(1)
        t_sp[...] = dot(st_ref[0:256, :], dot(wsp_ref[...], m1)) + dot(bsp_ref[...], m1)
        m2 = mcat(2)
        t_ab[...] = dot(ab_ref[...], dot(wab_ref[...], m2)) + dot(bab_ref[...], m2)

        m3 = mcat(3)
        item_part = (dot(it_ref[...], dot(wit_ref[0:128, :], m3))
                     + dot(bit_ref[...], m3))
        fxm = dot(wit_ref[128:144, :], m3)
        t_itfx[...] = jnp.concatenate(
            [item_part + fxm[f:f + 1, :] for f in range(16)] + [item_part],
            axis=0)

        mv_f[...] = dot(mv_ref[0:256, :], dot(wmv_ref[0:256, :], m4)) + dot(bmv_ref[...], m4)
        r64 = lax.broadcasted_iota(jnp.int32, (64, 1), 0)
        pp_blk[...] = dot(_bitsf(r64, 6), dot(wmv_ref[256:262, :], m4))

        t_r[...] = jnp.concatenate(
            [dot(woh_ref[0:1, :], m0), jnp.zeros((7, DD), jnp.float32)], axis=0)

    mvf = mv_f[...]
    t_mvpp[...] = jnp.concatenate(
        [mvf + pp_blk[pl.ds(8 * step + j, 1), :] for j in range(8)], axis=0)


def _fuse_tables(p):
    f32 = jnp.float32
    shapes = [(1024, DD), (128, DD), (864, DD), (405, DD), (256, DD),
              (256, DD), (4352, DD), (MV_CHUNK * MV_STEPS, DD), (8, DD)]
    outs = [jax.ShapeDtypeStruct(s, f32) for s in shapes]
    nil = lambda i: (0, 0)
    out_specs = [pl.BlockSpec(s, nil) for s in shapes[:-2]] + [
        pl.BlockSpec((MV_CHUNK, DD), lambda i: (i, 0)),
        pl.BlockSpec((8, DD), nil)]
    return pl.pallas_call(
        _fuse_body,
        grid=(MV_STEPS,),
        in_specs=[
            pl.BlockSpec(a.shape, functools.partial(lambda n, i: (0,) * n,
                                                    len(a.shape)))
            for a in _fuse_args(p)],
        out_specs=out_specs,
        out_shape=tuple(outs),
        scratch_shapes=[pltpu.VMEM((256, DD), f32), pltpu.VMEM((64, DD), f32)],
    )(*_fuse_args(p))


def _fuse_args(p):
    return (p['species_table'], p['ability_table'], p['item_table'],
            p['move_table'],
            p['W_hp'], p['W_level'], p['W_active'], p['W_onehot'],
            p['W_species'], p['W_ability'], p['W_item'], p['W_moves'],
            p['b_hp'][None, :], p['b_level'][None, :], p['b_active'][None, :],
            p['b_onehot'][None, :],
            p['b_species'][None, :], p['b_ability'][None, :],
            p['b_item'][None, :], p['b_moves'][None, :],
            p['Wg'], p['Wv'])


S = 16
NW = 32
NA_W = 4096 // NW
NS_W = 24576 // NW
B_HP, B_LV, B_GSB, B_TSFA, B_SP, B_AB, B_ITFX, B_MVPP = (
    0, 1024, 1152, 2016, 2421, 2677, 2933, 7285)


def _sc_body(act_hbm, side_hbm, t_all, t_r,
             out_a, out_s,
             e_act, e_side, idx_a, idx_b, d_a, d_b,
             r_v, rat_v, out_v0, out_v1, sem, sem_o):
    wid = lax.axis_index("s") * 2 + lax.axis_index("c")
    pltpu.sync_copy(t_r.at[pl.ds(0, DD)], r_v)
    pltpu.sync_copy(act_hbm.at[pl.ds(wid * NA_W * 24, NA_W * 24)], e_act)
    pltpu.sync_copy(side_hbm.at[pl.ds(wid * NS_W * 24, NS_W * 24)], e_side)
    lanes = lax.iota(jnp.int32, S)
    rg = [r_v[pl.ds(s * 16, 16)] for s in range(16)]
    rv = [r_v[pl.ds(256 + s * 16, 16)] for s in range(16)]

    def do_chunk(e_v, out_hbm, base, k, out_v):
        def feat(c):
            return plsc.load_gather(e_v, [lanes * 24 + (k * (S * 24) + c)])

        hp = feat(0)
        mx = jnp.maximum(feat(1), 1)
        ratio = jnp.clip(hp.astype(jnp.float32) / mx.astype(jnp.float32),
                         0.0, 1.0)
        rat_v[...] = ratio
        i_hp = jnp.minimum((1023 * hp) // mx, 1023)
        i_lv = jnp.bitwise_and(feat(10), 127)
        g3 = jnp.minimum(feat(2), 3)
        s7 = jnp.minimum(feat(3), 7)
        b2 = jnp.minimum(feat(4), 2)
        t2 = jnp.minimum(feat(5), 2)
        n2 = jnp.minimum(feat(6), 2)
        i_gsb = (((g3 * 8 + s7) * 3 + b2) * 3 + t2) * 3 + n2
        tox = jnp.minimum(feat(7), 8)
        slp = jnp.minimum(feat(8), 4)
        fnt = jnp.minimum(feat(9), 2)
        act = jnp.minimum(feat(11), 2)
        i_tsf = ((tox * 5 + slp) * 3 + fnt) * 3 + act
        idx_a[pl.ds(0, S)] = B_HP + i_hp
        idx_a[pl.ds(S, S)] = B_LV + i_lv
        idx_a[pl.ds(2 * S, S)] = B_GSB + i_gsb
        idx_a[pl.ds(3 * S, S)] = B_TSFA + i_tsf
        idx_a[pl.ds(4 * S, S)] = B_SP + feat(12)
        idx_a[pl.ds(5 * S, S)] = B_AB + feat(13)
        idx_b[pl.ds(0, S)] = B_ITFX + jnp.minimum(feat(15), 16) * 256 + feat(14)
        for j in range(4):
            idx_b[pl.ds((1 + j) * S, S)] = (
                B_MVPP + jnp.bitwise_and(feat(20 + j), 63) * 256 + feat(16 + j))

        ha = pltpu.async_copy(t_all.at[idx_a], d_a, sem)
        hb = pltpu.async_copy(t_all.at[idx_b], d_b, sem)
        ha.wait()
        hb.wait()

        def ent(i, carry):
            ri = plsc.load_gather(rat_v, [jnp.broadcast_to(i, (S,))])
            for s in range(16):
                cg = s * 16
                cv = cg + 256
                g0 = (d_a[i, pl.ds(cg, 16)]
                      + d_a[i + S, pl.ds(cg, 16)]
                      + d_a[i + 2 * S, pl.ds(cg, 16)]
                      + d_a[i + 3 * S, pl.ds(cg, 16)]
                      + ri * rg[s])
                v0 = (d_a[i, pl.ds(cv, 16)]
                      + d_a[i + S, pl.ds(cv, 16)]
                      + d_a[i + 2 * S, pl.ds(cv, 16)]
                      + d_a[i + 3 * S, pl.ds(cv, 16)]
                      + ri * rv[s])
                g1 = d_a[i + 4 * S, pl.ds(cg, 16)]
                v1 = d_a[i + 4 * S, pl.ds(cv, 16)]
                g2 = d_a[i + 5 * S, pl.ds(cg, 16)]
                v2 = d_a[i + 5 * S, pl.ds(cv, 16)]
                g3_ = d_b[i, pl.ds(cg, 16)]
                v3 = d_b[i, pl.ds(cv, 16)]
                g4 = (d_b[i + S, pl.ds(cg, 16)]
                      + d_b[i + 2 * S, pl.ds(cg, 16)]
                      + d_b[i + 3 * S, pl.ds(cg, 16)]
                      + d_b[i + 4 * S, pl.ds(cg, 16)])
                v4 = (d_b[i + S, pl.ds(cv, 16)]
                      + d_b[i + 2 * S, pl.ds(cv, 16)]
                      + d_b[i + 3 * S, pl.ds(cv, 16)]
                      + d_b[i + 4 * S, pl.ds(cv, 16)])
                m = jnp.maximum(jnp.maximum(jnp.maximum(g0, g1),
                                            jnp.maximum(g2, g3_)), g4)
                e0 = jnp.exp(g0 - m)
                e1 = jnp.exp(g1 - m)
                e2 = jnp.exp(g2 - m)
                e3 = jnp.exp(g3_ - m)
                e4 = jnp.exp(g4 - m)
                den = e0 + e1 + e2 + e3 + e4
                num = e0 * v0 + e1 * v1 + e2 * v2 + e3 * v3 + e4 * v4
                out_v[i, pl.ds(cg, 16)] = num / den
            return carry

        lax.fori_loop(0, S, ent, 0)
        return pltpu.async_copy(out_v, out_hbm.at[pl.ds(base + k * S, S)],
                                sem_o)

    def loops(e_v, out_hbm, base, nchunk):
        def pair(p, c):
            h0 = do_chunk(e_v, out_hbm, base, 2 * p, out_v0)
            h1 = do_chunk(e_v, out_hbm, base, 2 * p + 1, out_v1)
            h0.wait()
            h1.wait()
            return c

        lax.fori_loop(0, nchunk // 2, pair, 0)

    loops(e_act, out_a, wid * NA_W, NA_W // S)
    loops(e_side, out_s, wid * NS_W, NS_W // S)


def kernel(active_entities, side_entities, params):
    B = active_entities.shape[0]
    act = active_entities.reshape(-1, 24)
    side = side_entities.reshape(-1, 24)
    NA, NS = act.shape[0], side.shape[0]
    tables = _fuse_tables(params)

    f32 = jnp.float32
    mesh = plsc.VectorSubcoreMesh(core_axis_name="c", subcore_axis_name="s",
                                  num_cores=2, num_subcores=16)
    sc = functools.partial(
        pl.kernel,
        out_type=[jax.ShapeDtypeStruct((NA, D), f32),
                  jax.ShapeDtypeStruct((NS, D), f32)],
        mesh=mesh,
        compiler_params=pltpu.CompilerParams(needs_layout_passes=False),
        scratch_types=[pltpu.VMEM((NA_W * 24,), jnp.int32),
                       pltpu.VMEM((NS_W * 24,), jnp.int32),
                       pltpu.VMEM((6 * S,), jnp.int32),
                       pltpu.VMEM((5 * S,), jnp.int32),
                       pltpu.VMEM((6 * S, DD), f32),
                       pltpu.VMEM((5 * S, DD), f32),
                       pltpu.VMEM((DD,), f32),
                       pltpu.VMEM((S,), f32),
                       pltpu.VMEM((S, D), f32),
                       pltpu.VMEM((S, D), f32),
                       pltpu.SemaphoreType.DMA,
                       pltpu.SemaphoreType.DMA],
    )(_sc_body)
    t_all = jnp.concatenate(tables[:-1], axis=0)
    out_a, out_s = sc(act.reshape(-1), side.reshape(-1),
                      t_all, tables[-1].reshape(-1))

    active_embeddings = out_a.reshape(B, -1, D)
    side_embeddings = out_s.reshape(B, -1, D)
    tok = side_entities[..., 12]
    valid_team_mask = (tok != 0) | (tok != 1)
    return active_embeddings, side_embeddings, valid_team_mask


# speedup vs baseline: 6.4820x; 6.4820x over previous
import functools

import jax
import jax.numpy as jnp
from jax.experimental import pallas as pl
from jax.experimental.pallas import tpu as pltpu

ENTITY_SIZE = 256
HP, MAXHP, GENDER, STATUS, BCB, TRAPPED, NEWSW, TOXIC, SLEEP, FAINTED, LEVEL, ACTIVE, SPECIES, ABILITY, ITEM, ITEM_EFFECT = range(16)


def _fuse_body(st_ref, ab_ref, it_ref, mv_ref,
               whp_ref, wlv_ref, wac_ref, woh_ref,
               wsp_ref, wab_ref, wit_ref, wmv_ref,
               bhp_ref, blv_ref, bac_ref, boh_ref,
               bsp_ref, bab_ref, bit_ref, bmv_ref,
               wg_ref, wv_ref,
               f0g, f1g, f2g, f3ig, f3fg, f4mg, f4pg,
               f0v, f1v, f2v, f3iv, f3fv, f4mv, f4pv,
               biasg, biasv):
    w0 = jnp.concatenate(
        [whp_ref[...], wlv_ref[...], wac_ref[...], woh_ref[...]], axis=0)
    b0 = bhp_ref[...] + blv_ref[...] + bac_ref[...] + boh_ref[...]

    def dot(a, b):
        return jnp.dot(a, b, preferred_element_type=jnp.float32)

    for (mref, fg0, fg1, fg2, fg3i, fg3f, fg4m, fg4p, bias) in (
            (wg_ref, f0g, f1g, f2g, f3ig, f3fg, f4mg, f4pg, biasg),
            (wv_ref, f0v, f1v, f2v, f3iv, f3fv, f4mv, f4pv, biasv)):
        m0 = mref[0]
        m1 = mref[1]
        m2 = mref[2]
        m3 = mref[3]
        m4 = mref[4]
        bf = jnp.bfloat16
        fg0[...] = dot(w0, m0).astype(bf)
        fg1[...] = dot(st_ref[0:256, :], dot(wsp_ref[...], m1)).astype(bf)
        fg2[...] = dot(ab_ref[...], dot(wab_ref[...], m2)).astype(bf)
        fg3i[...] = dot(it_ref[...], dot(wit_ref[0:128, :], m3)).astype(bf)
        fg3f[...] = dot(wit_ref[128:144, :], m3).astype(bf)
        fg4m[...] = dot(mv_ref[0:256, :], dot(wmv_ref[0:256, :], m4)).astype(bf)
        fg4p[...] = dot(wmv_ref[256:262, :], m4).astype(bf)
        bias[...] = jnp.concatenate([
            dot(b0, m0),
            dot(bsp_ref[...], m1),
            dot(bab_ref[...], m2),
            dot(bit_ref[...], m3),
            4.0 * dot(bmv_ref[...], m4),
        ], axis=0)


def _fuse_tables(p):
    D = ENTITY_SIZE
    f32 = jnp.float32
    outs = (
        [jax.ShapeDtypeStruct(s, jnp.bfloat16) for s in
         ((50, D), (256, D), (256, D), (256, D), (16, D), (256, D), (6, D))] * 2
        + [jax.ShapeDtypeStruct((5, D), f32)] * 2)
    return pl.pallas_call(
        _fuse_body,
        out_shape=tuple(outs),
    )(p['species_table'], p['ability_table'], p['item_table'], p['move_table'],
      p['W_hp'], p['W_level'], p['W_active'], p['W_onehot'],
      p['W_species'], p['W_ability'], p['W_item'], p['W_moves'],
      p['b_hp'][None, :], p['b_level'][None, :], p['b_active'][None, :],
      p['b_onehot'][None, :],
      p['b_species'][None, :], p['b_ability'][None, :], p['b_item'][None, :],
      p['b_moves'][None, :],
      p['Wg'], p['Wv'])


def _oh(x, n, blk):
    i = jax.lax.broadcasted_iota(jnp.int32, (blk, n), 1)
    return (i == x).astype(jnp.bfloat16)


def _bits(x, nbits, blk):
    i = jax.lax.broadcasted_iota(jnp.int32, (blk, nbits), 1)
    mask = jnp.left_shift(jnp.int32(1), i)
    return (jnp.bitwise_and(x, mask) != 0).astype(jnp.bfloat16)


def _main_body(e_ref,
               f0g, f1g, f2g, f3ig, f3fg, f4mg, f4pg,
               f0v, f1v, f2v, f3iv, f3fv, f4mv, f4pv,
               biasg, biasv,
               out_ref, *, blk):
    e = e_ref[...]
    hp = e[:, 0:1].astype(jnp.float32)
    maxhp = jnp.maximum(e[:, 1:2].astype(jnp.float32), 1.0)
    ratio = jnp.clip(hp / maxhp, 0.0, 1.0)
    hp_token = (1023.0 * ratio).astype(jnp.int32)

    feat0 = jnp.concatenate([
        _bits(hp_token, 10, blk),
        _bits(e[:, 10:11], 7, blk),
        _oh(e[:, 11:12], 2, blk),
        ratio.astype(jnp.bfloat16),
        _oh(e[:, 2:3], 3, blk),
        _oh(e[:, 3:4], 7, blk),
        _oh(e[:, 4:5], 2, blk),
        _oh(e[:, 5:6], 2, blk),
        _oh(e[:, 6:7], 2, blk),
        _oh(e[:, 7:8], 8, blk),
        _oh(e[:, 8:9], 4, blk),
        _oh(e[:, 9:10], 2, blk),
    ], axis=1)
    oh_sp = _oh(e[:, 12:13], 256, blk)
    oh_ab = _oh(e[:, 13:14], 256, blk)
    oh_it = _oh(e[:, 14:15], 256, blk)
    oh_fx = _oh(e[:, 15:16], 16, blk)
    mh_mv = (_oh(e[:, 16:17], 256, blk) + _oh(e[:, 17:18], 256, blk)
             + _oh(e[:, 18:19], 256, blk) + _oh(e[:, 19:20], 256, blk))
    pp6 = (_bits(e[:, 20:21], 6, blk) + _bits(e[:, 21:22], 6, blk)
           + _bits(e[:, 22:23], 6, blk) + _bits(e[:, 23:24], 6, blk))

    def dot(a, b):
        return jnp.dot(a, b, preferred_element_type=jnp.float32)

    bg = biasg[...]
    bv = biasv[...]
    g = [dot(feat0, f0g[...]) + bg[0:1],
         dot(oh_sp, f1g[...]) + bg[1:2],
         dot(oh_ab, f2g[...]) + bg[2:3],
         dot(oh_it, f3ig[...]) + dot(oh_fx, f3fg[...]) + bg[3:4],
         dot(mh_mv, f4mg[...]) + dot(pp6, f4pg[...]) + bg[4:5]]
    v = [dot(feat0, f0v[...]) + bv[0:1],
         dot(oh_sp, f1v[...]) + bv[1:2],
         dot(oh_ab, f2v[...]) + bv[2:3],
         dot(oh_it, f3iv[...]) + dot(oh_fx, f3fv[...]) + bv[3:4],
         dot(mh_mv, f4mv[...]) + dot(pp6, f4pv[...]) + bv[4:5]]

    m = jnp.maximum(jnp.maximum(jnp.maximum(g[0], g[1]),
                                jnp.maximum(g[2], g[3])), g[4])
    es = [jnp.exp(gi - m) for gi in g]
    denom = es[0] + es[1] + es[2] + es[3] + es[4]
    num = es[0] * v[0] + es[1] * v[1] + es[2] * v[2] + es[3] * v[3] + es[4] * v[4]
    out_ref[...] = num / denom


def _encode(e, tables, blk):
    n = e.shape[0]
    return pl.pallas_call(
        functools.partial(_main_body, blk=blk),
        grid=(n // blk,),
        in_specs=[pl.BlockSpec((blk, 24), lambda i: (i, 0))]
        + [pl.BlockSpec(t.shape, lambda i: (0, 0)) for t in tables],
        out_specs=pl.BlockSpec((blk, ENTITY_SIZE), lambda i: (i, 0)),
        out_shape=jax.ShapeDtypeStruct((n, ENTITY_SIZE), jnp.float32),
    )(e, *tables)


def kernel(active_entities, side_entities, params):
    B = active_entities.shape[0]
    tables = _fuse_tables(params)
    out_a = _encode(active_entities.reshape(-1, 24), tables, 1024)
    out_s = _encode(side_entities.reshape(-1, 24), tables, 1024)
    active_embeddings = out_a.reshape(B, -1, ENTITY_SIZE)
    side_embeddings = out_s.reshape(B, -1, ENTITY_SIZE)
    tok = side_entities[..., SPECIES]
    valid_team_mask = (tok != 0) | (tok != 1)
    return active_embeddings, side_embeddings, valid_team_mask


# speedup vs baseline: 6.5295x; 1.0073x over previous
import functools

import jax
import jax.numpy as jnp
from jax.experimental import pallas as pl
from jax.experimental.pallas import tpu as pltpu

ENTITY_SIZE = 256
HP, MAXHP, GENDER, STATUS, BCB, TRAPPED, NEWSW, TOXIC, SLEEP, FAINTED, LEVEL, ACTIVE, SPECIES, ABILITY, ITEM, ITEM_EFFECT = range(16)


def _fuse_body(st_ref, ab_ref, it_ref, mv_ref,
               whp_ref, wlv_ref, wac_ref, woh_ref,
               wsp_ref, wab_ref, wit_ref, wmv_ref,
               bhp_ref, blv_ref, bac_ref, boh_ref,
               bsp_ref, bab_ref, bit_ref, bmv_ref,
               wg_ref, wv_ref,
               f0g, f1g, f2g, f3ig, f3fg, f4mg, f4pg,
               f0v, f1v, f2v, f3iv, f3fv, f4mv, f4pv,
               biasg, biasv):
    w0 = jnp.concatenate(
        [whp_ref[...], wlv_ref[...], wac_ref[...], woh_ref[...]], axis=0)
    b0 = bhp_ref[...] + blv_ref[...] + bac_ref[...] + boh_ref[...]

    def dot(a, b):
        return jnp.dot(a, b, preferred_element_type=jnp.float32)

    for (mref, fg0, fg1, fg2, fg3i, fg3f, fg4m, fg4p, bias) in (
            (wg_ref, f0g, f1g, f2g, f3ig, f3fg, f4mg, f4pg, biasg),
            (wv_ref, f0v, f1v, f2v, f3iv, f3fv, f4mv, f4pv, biasv)):
        m0 = mref[0]
        m1 = mref[1]
        m2 = mref[2]
        m3 = mref[3]
        m4 = mref[4]
        bf = jnp.bfloat16
        fg0[...] = dot(w0, m0).astype(bf)
        fg1[...] = dot(st_ref[0:256, :], dot(wsp_ref[...], m1)).astype(bf)
        fg2[...] = dot(ab_ref[...], dot(wab_ref[...], m2)).astype(bf)
        fg3i[...] = dot(it_ref[...], dot(wit_ref[0:128, :], m3)).astype(bf)
        fg3f[...] = dot(wit_ref[128:144, :], m3).astype(bf)
        fg4m[...] = dot(mv_ref[0:256, :], dot(wmv_ref[0:256, :], m4)).astype(bf)
        fg4p[...] = dot(wmv_ref[256:262, :], m4).astype(bf)
        bias[...] = jnp.concatenate([
            dot(b0, m0),
            dot(bsp_ref[...], m1),
            dot(bab_ref[...], m2),
            dot(bit_ref[...], m3),
            4.0 * dot(bmv_ref[...], m4),
        ], axis=0)


def _fuse_tables(p):
    D = ENTITY_SIZE
    f32 = jnp.float32
    outs = (
        [jax.ShapeDtypeStruct(s, jnp.bfloat16) for s in
         ((50, D), (256, D), (256, D), (256, D), (16, D), (256, D), (6, D))] * 2
        + [jax.ShapeDtypeStruct((5, D), f32)] * 2)
    return pl.pallas_call(
        _fuse_body,
        out_shape=tuple(outs),
    )(p['species_table'], p['ability_table'], p['item_table'], p['move_table'],
      p['W_hp'], p['W_level'], p['W_active'], p['W_onehot'],
      p['W_species'], p['W_ability'], p['W_item'], p['W_moves'],
      p['b_hp'][None, :], p['b_level'][None, :], p['b_active'][None, :],
      p['b_onehot'][None, :],
      p['b_species'][None, :], p['b_ability'][None, :], p['b_item'][None, :],
      p['b_moves'][None, :],
      p['Wg'], p['Wv'])


def _oh(x, n, blk):
    i = jax.lax.broadcasted_iota(jnp.int32, (blk, n), 1)
    return (i == x).astype(jnp.bfloat16)


def _bits(x, nbits, blk):
    i = jax.lax.broadcasted_iota(jnp.int32, (blk, nbits), 1)
    mask = jnp.left_shift(jnp.int32(1), i)
    return (jnp.bitwise_and(x, mask) != 0).astype(jnp.bfloat16)


def _main_body(e_ref,
               f0g, f1g, f2g, f3ig, f3fg, f4mg, f4pg,
               f0v, f1v, f2v, f3iv, f3fv, f4mv, f4pv,
               biasg, biasv,
               out_ref, *, blk):
    e = e_ref[...]
    hp = e[:, 0:1].astype(jnp.float32)
    maxhp = jnp.maximum(e[:, 1:2].astype(jnp.float32), 1.0)
    ratio = jnp.clip(hp / maxhp, 0.0, 1.0)
    hp_token = (1023.0 * ratio).astype(jnp.int32)

    feat0 = jnp.concatenate([
        _bits(hp_token, 10, blk),
        _bits(e[:, 10:11], 7, blk),
        _oh(e[:, 11:12], 2, blk),
        ratio.astype(jnp.bfloat16),
        _oh(e[:, 2:3], 3, blk),
        _oh(e[:, 3:4], 7, blk),
        _oh(e[:, 4:5], 2, blk),
        _oh(e[:, 5:6], 2, blk),
        _oh(e[:, 6:7], 2, blk),
        _oh(e[:, 7:8], 8, blk),
        _oh(e[:, 8:9], 4, blk),
        _oh(e[:, 9:10], 2, blk),
    ], axis=1)
    oh_sp = _oh(e[:, 12:13], 256, blk)
    oh_ab = _oh(e[:, 13:14], 256, blk)
    oh_it = _oh(e[:, 14:15], 256, blk)
    oh_fx = _oh(e[:, 15:16], 16, blk)
    mh_mv = (_oh(e[:, 16:17], 256, blk) + _oh(e[:, 17:18], 256, blk)
             + _oh(e[:, 18:19], 256, blk) + _oh(e[:, 19:20], 256, blk))
    pp6 = (_bits(e[:, 20:21], 6, blk) + _bits(e[:, 21:22], 6, blk)
           + _bits(e[:, 22:23], 6, blk) + _bits(e[:, 23:24], 6, blk))

    def dot(a, b):
        return jnp.dot(a, b, preferred_element_type=jnp.float32)

    bg = biasg[...]
    bv = biasv[...]
    g = [dot(feat0, f0g[...]) + bg[0:1],
         dot(oh_sp, f1g[...]) + bg[1:2],
         dot(oh_ab, f2g[...]) + bg[2:3],
         dot(oh_it, f3ig[...]) + dot(oh_fx, f3fg[...]) + bg[3:4],
         dot(mh_mv, f4mg[...]) + dot(pp6, f4pg[...]) + bg[4:5]]
    v = [dot(feat0, f0v[...]) + bv[0:1],
         dot(oh_sp, f1v[...]) + bv[1:2],
         dot(oh_ab, f2v[...]) + bv[2:3],
         dot(oh_it, f3iv[...]) + dot(oh_fx, f3fv[...]) + bv[3:4],
         dot(mh_mv, f4mv[...]) + dot(pp6, f4pv[...]) + bv[4:5]]

    m = jnp.maximum(jnp.maximum(jnp.maximum(g[0], g[1]),
                                jnp.maximum(g[2], g[3])), g[4])
    es = [jnp.exp(gi - m) for gi in g]
    denom = es[0] + es[1] + es[2] + es[3] + es[4]
    num = es[0] * v[0] + es[1] * v[1] + es[2] * v[2] + es[3] * v[3] + es[4] * v[4]
    out_ref[...] = num / denom


def _encode(e, tables, blk):
    n = e.shape[0]
    return pl.pallas_call(
        functools.partial(_main_body, blk=blk),
        grid=(n // blk,),
        in_specs=[pl.BlockSpec((blk, 24), lambda i: (i, 0))]
        + [pl.BlockSpec(t.shape, lambda i: (0, 0)) for t in tables],
        out_specs=pl.BlockSpec((blk, ENTITY_SIZE), lambda i: (i, 0)),
        out_shape=jax.ShapeDtypeStruct((n, ENTITY_SIZE), jnp.float32),
    )(e, *tables)


def kernel(active_entities, side_entities, params):
    B = active_entities.shape[0]
    tables = _fuse_tables(params)
    out_a = _encode(active_entities.reshape(-1, 24), tables, 2048)
    out_s = _encode(side_entities.reshape(-1, 24), tables, 2048)
    active_embeddings = out_a.reshape(B, -1, ENTITY_SIZE)
    side_embeddings = out_s.reshape(B, -1, ENTITY_SIZE)
    tok = side_entities[..., SPECIES]
    valid_team_mask = (tok != 0) | (tok != 1)
    return active_embeddings, side_embeddings, valid_team_mask
